# baseline (speedup 1.0000x reference)
import itertools
import math

import jax
import jax.numpy as jnp
import numpy as np
from jax.experimental import pallas as pl
from jax.experimental.pallas import tpu as pltpu

DIM = 4096
N_MODULES = 16
N_ACTIVE = 2
_SUBSETS_T_NP = np.array(
    list(itertools.combinations(range(N_MODULES), N_ACTIVE)), dtype=np.int32
).T
N_SUBSETS = _SUBSETS_T_NP.shape[1]

SEQ_BLOCK = 256


def _ctrl_kernel(x_ref, pm_ref, w_ref, b_ref, subs_ref,
                 logp_ref, sel_ref, pred_ref, acc_ref):
    i = pl.program_id(0)
    nb = pl.num_programs(0)

    @pl.when(i == 0)
    def _init():
        acc_ref[...] = jnp.zeros_like(acc_ref)

    pm = pm_ref[:, pl.ds(i * SEQ_BLOCK, SEQ_BLOCK)]
    m = jnp.where(pm, 0.0, 1.0).astype(jnp.float32)
    acc_ref[...] += jnp.sum(x_ref[...] * m[:, :, None], axis=1)

    @pl.when(i == nb - 1)
    def _final():
        xs = acc_ref[...]
        logits = jax.lax.dot_general(
            xs, w_ref[...], (((1,), (1,)), ((), ())),
            preferred_element_type=jnp.float32) + b_ref[...]
        mx = jnp.max(logits, axis=-1, keepdims=True)
        sh = logits - mx
        lse = jnp.log(jnp.sum(jnp.exp(sh), axis=-1, keepdims=True))
        logp_ref[...] = (sh - lse)[:, None, :]

        ids = jax.lax.broadcasted_iota(jnp.int32, logits.shape, 1)
        pred = jnp.min(
            jnp.where(logits == mx, ids, jnp.int32(N_SUBSETS)),
            axis=-1, keepdims=True)
        pred_ref[...] = pred

        onehot = (ids == pred).astype(jnp.int32)
        sel_ref[...] = jnp.sum(
            onehot[:, None, :] * subs_ref[...][None, :, :], axis=-1)


def kernel(x, padding_mask, W, b):
    B, T, _ = x.shape
    nb = T // SEQ_BLOCK
    subs_t = jnp.asarray(_SUBSETS_T_NP)
    b2 = b.reshape(1, N_SUBSETS)

    logp, sel, pred = pl.pallas_call(
        _ctrl_kernel,
        grid=(nb,),
        in_specs=[
            pl.BlockSpec((B, SEQ_BLOCK, DIM), lambda i: (0, i, 0)),
            pl.BlockSpec((B, T), lambda i: (0, 0)),
            pl.BlockSpec((N_SUBSETS, DIM), lambda i: (0, 0)),
            pl.BlockSpec((1, N_SUBSETS), lambda i: (0, 0)),
            pl.BlockSpec((N_ACTIVE, N_SUBSETS), lambda i: (0, 0)),
        ],
        out_specs=[
            pl.BlockSpec((B, 1, N_SUBSETS), lambda i: (0, 0, 0)),
            pl.BlockSpec((B, N_ACTIVE), lambda i: (0, 0)),
            pl.BlockSpec((B, 1), lambda i: (0, 0)),
        ],
        out_shape=[
            jax.ShapeDtypeStruct((B, 1, N_SUBSETS), jnp.float32),
            jax.ShapeDtypeStruct((B, N_ACTIVE), jnp.int32),
            jax.ShapeDtypeStruct((B, 1), jnp.int32),
        ],
        scratch_shapes=[pltpu.VMEM((B, DIM), jnp.float32)],
    )(x, padding_mask, W, b2, subs_t)

    return (logp, sel, pred)


# speedup vs baseline: 1.0061x; 1.0061x over previous
import itertools
import math

import jax
import jax.numpy as jnp
import numpy as np
from jax.experimental import pallas as pl
from jax.experimental.pallas import tpu as pltpu

DIM = 4096
N_MODULES = 16
N_ACTIVE = 2
_SUBSETS_T_NP = np.array(
    list(itertools.combinations(range(N_MODULES), N_ACTIVE)), dtype=np.int32
).T
N_SUBSETS = _SUBSETS_T_NP.shape[1]

SEQ_BLOCK = 256


def _ctrl_kernel(x_ref, pm_ref, w_ref, b_ref, subs_ref,
                 logp_ref, sel_ref, pred_ref, acc_ref):
    i = pl.program_id(0)
    nb = pl.num_programs(0)

    @pl.when(i == 0)
    def _init():
        acc_ref[...] = jnp.zeros_like(acc_ref)

    pm = pm_ref[:, pl.ds(i * SEQ_BLOCK, SEQ_BLOCK)]
    m = 1.0 - pm.astype(jnp.float32)
    acc_ref[...] += jnp.sum(x_ref[...] * m[:, :, None], axis=1)

    @pl.when(i == nb - 1)
    def _final():
        xs = acc_ref[...]
        logits = jax.lax.dot_general(
            xs, w_ref[...], (((1,), (1,)), ((), ())),
            preferred_element_type=jnp.float32) + b_ref[...]
        mx = jnp.max(logits, axis=-1, keepdims=True)
        sh = logits - mx
        lse = jnp.log(jnp.sum(jnp.exp(sh), axis=-1, keepdims=True))
        logp_ref[...] = (sh - lse)[:, None, :]

        ids = jax.lax.broadcasted_iota(jnp.int32, logits.shape, 1)
        pred = jnp.min(
            jnp.where(logits == mx, ids, jnp.int32(N_SUBSETS)),
            axis=-1, keepdims=True)
        pred_ref[...] = pred

        onehot = (ids == pred).astype(jnp.int32)
        sel_ref[...] = jnp.sum(
            onehot[:, None, :] * subs_ref[...][None, :, :], axis=-1)


def kernel(x, padding_mask, W, b):
    B, T, _ = x.shape
    nb = T // SEQ_BLOCK
    pm8 = padding_mask.view(jnp.int8)
    subs_t = jnp.asarray(_SUBSETS_T_NP)
    b2 = b.reshape(1, N_SUBSETS)

    logp, sel, pred = pl.pallas_call(
        _ctrl_kernel,
        grid=(nb,),
        in_specs=[
            pl.BlockSpec((B, SEQ_BLOCK, DIM), lambda i: (0, i, 0)),
            pl.BlockSpec((B, T), lambda i: (0, 0)),
            pl.BlockSpec((N_SUBSETS, DIM), lambda i: (0, 0)),
            pl.BlockSpec((1, N_SUBSETS), lambda i: (0, 0)),
            pl.BlockSpec((N_ACTIVE, N_SUBSETS), lambda i: (0, 0)),
        ],
        out_specs=[
            pl.BlockSpec((B, 1, N_SUBSETS), lambda i: (0, 0, 0)),
            pl.BlockSpec((B, N_ACTIVE), lambda i: (0, 0)),
            pl.BlockSpec((B, 1), lambda i: (0, 0)),
        ],
        out_shape=[
            jax.ShapeDtypeStruct((B, 1, N_SUBSETS), jnp.float32),
            jax.ShapeDtypeStruct((B, N_ACTIVE), jnp.int32),
            jax.ShapeDtypeStruct((B, 1), jnp.int32),
        ],
        scratch_shapes=[pltpu.VMEM((B, DIM), jnp.float32)],
    )(x, pm8, W, b2, subs_t)

    return (logp, sel, pred)


# speedup vs baseline: 1.0779x; 1.0713x over previous
import itertools
import math

import jax
import jax.numpy as jnp
import numpy as np
from jax.experimental import pallas as pl
from jax.experimental.pallas import tpu as pltpu

DIM = 4096
N_MODULES = 16
N_ACTIVE = 2
_SUBSETS_T_NP = np.array(
    list(itertools.combinations(range(N_MODULES), N_ACTIVE)), dtype=np.int32
).T
N_SUBSETS = _SUBSETS_T_NP.shape[1]

SEQ_BLOCK = 256


def _ctrl_kernel(x_ref, w_ref, b_ref, subs_ref,
                 logp_ref, sel_ref, pred_ref, acc_ref):
    i = pl.program_id(0)
    nb = pl.num_programs(0)

    @pl.when(i == 0)
    def _init():
        acc_ref[...] = jnp.zeros_like(acc_ref)

    acc_ref[...] += jnp.sum(x_ref[...], axis=1)

    @pl.when(i == nb - 1)
    def _final():
        xs = acc_ref[...]
        logits = jax.lax.dot_general(
            xs, w_ref[...], (((1,), (1,)), ((), ())),
            preferred_element_type=jnp.float32) + b_ref[...]
        mx = jnp.max(logits, axis=-1, keepdims=True)
        sh = logits - mx
        lse = jnp.log(jnp.sum(jnp.exp(sh), axis=-1, keepdims=True))
        logp_ref[...] = (sh - lse)[:, None, :]

        ids = jax.lax.broadcasted_iota(jnp.int32, logits.shape, 1)
        pred = jnp.min(
            jnp.where(logits == mx, ids, jnp.int32(N_SUBSETS)),
            axis=-1, keepdims=True)
        pred_ref[...] = pred

        onehot = (ids == pred).astype(jnp.int32)
        sel_ref[...] = jnp.sum(
            onehot[:, None, :] * subs_ref[...][None, :, :], axis=-1)


def kernel(x, padding_mask, W, b):
    del padding_mask
    B, T, _ = x.shape
    nb = T // SEQ_BLOCK
    subs_t = jnp.asarray(_SUBSETS_T_NP)
    b2 = b.reshape(1, N_SUBSETS)

    logp, sel, pred = pl.pallas_call(
        _ctrl_kernel,
        grid=(nb,),
        in_specs=[
            pl.BlockSpec((B, SEQ_BLOCK, DIM), lambda i: (0, i, 0)),
            pl.BlockSpec((N_SUBSETS, DIM), lambda i: (0, 0)),
            pl.BlockSpec((1, N_SUBSETS), lambda i: (0, 0)),
            pl.BlockSpec((N_ACTIVE, N_SUBSETS), lambda i: (0, 0)),
        ],
        out_specs=[
            pl.BlockSpec((B, 1, N_SUBSETS), lambda i: (0, 0, 0)),
            pl.BlockSpec((B, N_ACTIVE), lambda i: (0, 0)),
            pl.BlockSpec((B, 1), lambda i: (0, 0)),
        ],
        out_shape=[
            jax.ShapeDtypeStruct((B, 1, N_SUBSETS), jnp.float32),
            jax.ShapeDtypeStruct((B, N_ACTIVE), jnp.int32),
            jax.ShapeDtypeStruct((B, 1), jnp.int32),
        ],
        scratch_shapes=[pltpu.VMEM((B, DIM), jnp.float32)],
    )(x, W, b2, subs_t)

    return (logp, sel, pred)
